# baseline (speedup 1.0000x reference)
import functools

import jax
import jax.numpy as jnp
from jax import lax
from jax.experimental import pallas as pl
from jax.experimental.pallas import tpu as pltpu
from jax.experimental.pallas import tpu_sc as plsc

N = 10000
D = 256
H = 128
NT = 32
PT = 320
N_PAD = NT * PT
E_RAW = 160000
E_TOT = E_RAW + N
E_CAP = 170496
CE = 256


def _linear(x, W, b, bm):
    M, K = x.shape
    Ho = W.shape[1]
    b2 = b.reshape(1, Ho)

    def body(x_ref, w_ref, b_ref, o_ref):
        o_ref[...] = jnp.dot(x_ref[...], w_ref[...],
                             preferred_element_type=jnp.float32) + b_ref[...]

    return pl.pallas_call(
        body,
        grid=(M // bm,),
        in_specs=[pl.BlockSpec((bm, K), lambda i: (i, 0)),
                  pl.BlockSpec((K, Ho), lambda i: (0, 0)),
                  pl.BlockSpec((1, Ho), lambda i: (0, 0))],
        out_specs=pl.BlockSpec((bm, Ho), lambda i: (i, 0)),
        out_shape=jax.ShapeDtypeStruct((M, Ho), jnp.float32),
    )(x, W, b2)


def _mm(x, W, bm):
    M, K = x.shape
    Ho = W.shape[1]

    def body(x_ref, w_ref, o_ref):
        o_ref[...] = jnp.dot(x_ref[...], w_ref[...],
                             preferred_element_type=jnp.float32)

    return pl.pallas_call(
        body,
        grid=(M // bm,),
        in_specs=[pl.BlockSpec((bm, K), lambda i: (i, 0)),
                  pl.BlockSpec((K, Ho), lambda i: (0, 0))],
        out_specs=pl.BlockSpec((bm, Ho), lambda i: (i, 0)),
        out_shape=jax.ShapeDtypeStruct((M, Ho), jnp.float32),
    )(x, W)


def _mm_sum2(a, b, W, bm):
    M, K = a.shape
    Ho = W.shape[1]

    def body(a_ref, b_ref, w_ref, o_ref):
        o_ref[...] = jnp.dot(a_ref[...] + b_ref[...], w_ref[...],
                             preferred_element_type=jnp.float32)

    return pl.pallas_call(
        body,
        grid=(M // bm,),
        in_specs=[pl.BlockSpec((bm, K), lambda i: (i, 0)),
                  pl.BlockSpec((bm, K), lambda i: (i, 0)),
                  pl.BlockSpec((K, Ho), lambda i: (0, 0))],
        out_specs=pl.BlockSpec((bm, Ho), lambda i: (i, 0)),
        out_shape=jax.ShapeDtypeStruct((M, Ho), jnp.float32),
    )(a, b, W)


def _final(y0, y1, W, b, bm):
    M = y0.shape[0]
    K = y0.shape[1]
    Ho = W.shape[1]
    b2 = b.reshape(1, Ho)

    def body(a_ref, b_ref, w_ref, bias_ref, fin_ref, loss_ref):
        f = jax.nn.sigmoid(a_ref[...] * b_ref[...])
        fin_ref[...] = f
        loss_ref[...] = jnp.dot(f, w_ref[...],
                                preferred_element_type=jnp.float32) + bias_ref[...]

    return pl.pallas_call(
        body,
        grid=(M // bm,),
        in_specs=[pl.BlockSpec((bm, K), lambda i: (i, 0)),
                  pl.BlockSpec((bm, K), lambda i: (i, 0)),
                  pl.BlockSpec((K, Ho), lambda i: (0, 0)),
                  pl.BlockSpec((1, Ho), lambda i: (0, 0))],
        out_specs=[pl.BlockSpec((bm, K), lambda i: (i, 0)),
                   pl.BlockSpec((bm, Ho), lambda i: (i, 0))],
        out_shape=[jax.ShapeDtypeStruct((M, K), jnp.float32),
                   jax.ShapeDtypeStruct((M, Ho), jnp.float32)],
    )(y0, y1, W, b2)


@functools.cache
def _make_spmm(ng):
    mesh = plsc.VectorSubcoreMesh(core_axis_name="c", subcore_axis_name="s")
    nc = 2

    @functools.partial(
        pl.kernel,
        out_type=jax.ShapeDtypeStruct((ng * N_PAD, H), jnp.float32),
        mesh=mesh,
        scratch_types=[
            pltpu.VMEM((CE,), jnp.int32),
            pltpu.VMEM((CE,), jnp.int32),
            pltpu.VMEM((CE,), jnp.float32),
            pltpu.VMEM((CE, H), jnp.float32),
            pltpu.VMEM((PT, H), jnp.float32),
            pltpu.VMEM((ng * NT + 16,), jnp.int32),
            pltpu.VMEM((ng * NT + 16,), jnp.int32),
            pltpu.SemaphoreType.DMA,
        ],
    )
    def spmm(table, srcs, dstls, nrms, cnts, offps, bias, out,
             idx_v, dstl_v, norm_v, rows_v, slab, cnt_v, offp_v, sem):
        wid = lax.axis_index("s") * nc + lax.axis_index("c")
        pltpu.sync_copy(cnts, cnt_v.at[pl.ds(0, ng * NT)])
        pltpu.sync_copy(offps, offp_v.at[pl.ds(0, ng * NT)])
        lanes = lax.iota(jnp.int32, 16)
        for g in range(ng):
            pltpu.sync_copy(bias, slab)
            cntt = cnt_v[pl.ds(g * NT + wid, 16)][0]
            off = offp_v[pl.ds(g * NT + wid, 16)][0]
            nch = (cntt + (CE - 1)) // CE

            def chunk(k, _):
                base = pl.multiple_of(off + k * CE, 8)
                pltpu.sync_copy(srcs.at[pl.ds(base, CE)], idx_v)
                pltpu.sync_copy(dstls.at[pl.ds(base, CE)], dstl_v)
                pltpu.sync_copy(nrms.at[pl.ds(base, CE)], norm_v)
                pltpu.async_copy(table.at[idx_v], rows_v, sem).wait()
                nv = jnp.minimum(CE, cntt - k * CE)
                ngrp = (nv + 15) // 16

                def grp(i, carry):
                    e0 = i * 16
                    dlv = dstl_v[pl.ds(e0, 16)]
                    nmv = norm_v[pl.ds(e0, 16)]
                    nmv = jnp.where(e0 + lanes < nv, nmv, 0.0)
                    for l in range(16):
                        dl = dlv[l]
                        nm = nmv[l]
                        for j in range(H // 16):
                            v = rows_v[e0 + l, pl.ds(j * 16, 16)] * nm
                            plsc.addupdate(slab.at[dl, pl.ds(j * 16, 16)], v)
                    return carry

                lax.fori_loop(0, ngrp, grp, 0)
                return _

            lax.fori_loop(0, nch, chunk, 0)
            pltpu.sync_copy(slab, out.at[pl.ds(g * N_PAD + wid * PT, PT)])

    return spmm


def _prep_edges(ei):
    loop = jnp.arange(N, dtype=jnp.int32)
    src = jnp.concatenate([ei[0].astype(jnp.int32), loop])
    dst = jnp.concatenate([ei[1].astype(jnp.int32), loop])
    deg = jnp.zeros((N,), jnp.float32).at[dst].add(1.0)
    dis = jnp.where(deg > 0, 1.0 / jnp.sqrt(deg), 0.0)
    norm = dis[src] * dis[dst]
    bucket = dst // PT
    order = jnp.argsort(bucket, stable=True)
    srcs = src[order]
    dsts = dst[order]
    norms = norm[order]
    buckets = bucket[order]
    cnt = jnp.zeros((NT,), jnp.int32).at[bucket].add(1)
    cntp = (cnt + 7) & ~7
    zero1 = jnp.zeros((1,), jnp.int32)
    offp = jnp.concatenate([zero1, jnp.cumsum(cntp)[:-1]])
    off = jnp.concatenate([zero1, jnp.cumsum(cnt)[:-1]])
    pos = offp[buckets] + (jnp.arange(E_TOT, dtype=jnp.int32) - off[buckets])
    SRC = jnp.zeros((E_CAP,), jnp.int32).at[pos].set(srcs)
    DSTL = jnp.zeros((E_CAP,), jnp.int32).at[pos].set(dsts - buckets * PT)
    NRM = jnp.zeros((E_CAP,), jnp.float32).at[pos].set(norms)
    return SRC, DSTL, NRM, cntp, offp


def _bias_full(b):
    return jnp.tile(b.reshape(1, H), (PT, 1))


def kernel(x0, x1, x2, edge_index0, edge_index1, edge_index2,
           W_fc1, b_fc1, W_c1, b_c1, W_c2, b_c2,
           W_d1, b_d1, W_d2, b_d2, W_fc2, b_fc2):
    eis = [edge_index0, edge_index1, edge_index2]
    preps = [_prep_edges(ei) for ei in eis]

    SRC_c = jnp.concatenate([p[0] + g * N for g, p in enumerate(preps)])
    DSTL_c = jnp.concatenate([p[1] for p in preps])
    NRM_c = jnp.concatenate([p[2] for p in preps])
    CNT_c = jnp.concatenate([p[3] for p in preps])
    OFFP_c = jnp.concatenate([p[4] + g * E_CAP for g, p in enumerate(preps)])

    spmm3 = _make_spmm(3)
    spmm1 = _make_spmm(1)

    x3 = jnp.concatenate([x0, x1, x2], axis=0)
    pre3 = _linear(x3, W_fc1, b_fc1, 1000)

    h = _mm(pre3, W_c1, 1000)
    a1 = spmm3(h, SRC_c, DSTL_c, NRM_c, CNT_c, OFFP_c, _bias_full(b_c1))
    h1 = a1.reshape(3, N_PAD, H)[:, :N, :].reshape(3 * N, H)
    h = _mm(h1, W_c2, 1000)
    a2 = spmm3(h, SRC_c, DSTL_c, NRM_c, CNT_c, OFFP_c, _bias_full(b_c2))
    normal = a2.reshape(3, N_PAD, H)[:, :N, :]

    bias_d1 = _bias_full(b_d1)
    bias_d2 = _bias_full(b_d2)

    def dconv(s_a, s_b, g):
        SRC, DSTL, NRM, CNT, OFFP = preps[g]
        hh = _mm_sum2(s_a, s_b, W_d1, 1000)
        t = spmm1(hh, SRC, DSTL, NRM, CNT, OFFP, bias_d1)[:N, :]
        hh = _mm(t, W_d2, 1000)
        return spmm1(hh, SRC, DSTL, NRM, CNT, OFFP, bias_d2)[:N, :]

    n0, n1, n2 = normal[0], normal[1], normal[2]
    y0 = dconv(n1, n2, 0)
    y1 = dconv(y0, n2, 1)
    y2 = dconv(y0, y1, 2)

    fin_feat, loss_embedding = _final(y0, y1, W_fc2, b_fc2, 1000)

    pre_x = tuple(pre3.reshape(3, N, H))
    return (pre_x, (y0, y1, y2), fin_feat, loss_embedding)


# speedup vs baseline: 1.5058x; 1.5058x over previous
import functools

import jax
import jax.numpy as jnp
from jax import lax
from jax.experimental import pallas as pl
from jax.experimental.pallas import tpu as pltpu
from jax.experimental.pallas import tpu_sc as plsc

N = 10000
D = 256
H = 128
N_PAD = 10240
BM = 640
E_TOT = 160000 + N
CH = 128
KPG = 48
EPT = KPG * CH
E_PAD = 32 * EPT
DW = 16
KD = 128
DEG_EPT = KD * CH
DEG_PAD = 32 * DEG_EPT


_MESH = plsc.VectorSubcoreMesh(core_axis_name="c", subcore_axis_name="s")


@functools.cache
def _make_spmm(ng):
    @functools.partial(
        pl.kernel,
        out_type=jax.ShapeDtypeStruct((2 * ng * N_PAD, H), jnp.float32),
        mesh=_MESH,
        scratch_types=[
            pltpu.VMEM((CH,), jnp.int32),
            pltpu.VMEM((CH,), jnp.int32),
            pltpu.VMEM((CH, H), jnp.float32),
            pltpu.VMEM_SHARED((N_PAD, H), jnp.float32),
            pltpu.SemaphoreType.DMA,
        ],
    )
    def spmm(table, srcs, dsts, zrows, out, idx_s, idx_d, rows_v, acc, sem):
        c = lax.axis_index("c")
        s = lax.axis_index("s")
        wid = s * 2 + c
        stripe = pl.multiple_of(s * (N_PAD // 16), N_PAD // 16)
        for g in range(ng):
            pltpu.sync_copy(zrows, acc.at[pl.ds(stripe, N_PAD // 16)])
            base0 = (g * 32 + wid) * EPT
            plsc.subcore_barrier()

            @pl.loop(0, KPG)
            def chunk(j):
                b = pl.multiple_of(base0 + j * CH, CH)
                pltpu.sync_copy(srcs.at[pl.ds(b, CH)], idx_s)
                pltpu.sync_copy(dsts.at[pl.ds(b, CH)], idx_d)
                pltpu.async_copy(table.at[idx_s], rows_v, sem).wait()
                pltpu.sync_copy(rows_v, acc.at[idx_d], add=True)

            plsc.subcore_barrier()
            ob = pl.multiple_of((c * ng + g) * N_PAD + stripe, N_PAD // 16)
            pltpu.sync_copy(acc.at[pl.ds(stripe, N_PAD // 16)],
                            out.at[pl.ds(ob, N_PAD // 16)])
            if g + 1 < ng:
                plsc.subcore_barrier()

    return spmm


def _make_deg():
    rows = 3 * N_PAD

    @functools.partial(
        pl.kernel,
        out_type=jax.ShapeDtypeStruct((32 * rows,), jnp.float32),
        mesh=_MESH,
        scratch_types=[
            pltpu.VMEM((CH,), jnp.int32),
            pltpu.VMEM((rows,), jnp.float32),
        ],
        compiler_params=pltpu.CompilerParams(needs_layout_passes=False),
    )
    def deg(dsts, zeros, out, idx_d, acc):
        c = lax.axis_index("c")
        s = lax.axis_index("s")
        wid = s * 2 + c
        pltpu.sync_copy(zeros, acc)
        base0 = wid * DEG_EPT
        one = jnp.full((16,), 1.0, jnp.float32)

        @pl.loop(0, KD)
        def chunk(j):
            b = pl.multiple_of(base0 + j * CH, CH)
            pltpu.sync_copy(dsts.at[pl.ds(b, CH)], idx_d)
            for k in range(CH // 16):
                vec = idx_d[pl.ds(k * 16, 16)]
                plsc.addupdate_scatter(acc, [vec], one)

        ob = pl.multiple_of(wid * rows, rows)
        pltpu.sync_copy(acc, out.at[pl.ds(ob, rows)])

    return deg


def _dis(deg_blk):
    d = deg_blk[:, 0:1]
    return jnp.where(d > 0, 1.0 / jnp.sqrt(d), 0.0)


def _blk(h, off):
    return pl.BlockSpec((BM, h), lambda i, o=off: (i + o, 0))


def _wblk(k, h):
    return pl.BlockSpec((k, h), lambda i: (0, 0))


def _tc_a(x3, degp, W_fc1, b_fc1, W_c1):
    M = 3 * N_PAD

    def body(x_ref, d_ref, w1_ref, b1_ref, wc_ref,
             pre_ref, xs_ref, deg_ref):
        ds = jnp.sum(d_ref[...].T, axis=1, keepdims=True)
        deg_ref[...] = jnp.broadcast_to(ds, (BM, DW))
        pre = jnp.dot(x_ref[...], w1_ref[...],
                      preferred_element_type=jnp.float32) + b1_ref[...]
        pre_ref[...] = pre
        dis = jnp.where(ds > 0, 1.0 / jnp.sqrt(ds), 0.0)
        xs_ref[...] = dis * jnp.dot(pre, wc_ref[...],
                                    preferred_element_type=jnp.float32)

    return pl.pallas_call(
        body,
        grid=(M // BM,),
        in_specs=[_blk(D, 0), pl.BlockSpec((32, BM), lambda i: (0, i)),
                  _wblk(D, H), _wblk(1, H), _wblk(H, H)],
        out_specs=[_blk(H, 0), _blk(H, 0), _blk(DW, 0)],
        out_shape=[jax.ShapeDtypeStruct((M, H), jnp.float32),
                   jax.ShapeDtypeStruct((M, H), jnp.float32),
                   jax.ShapeDtypeStruct((M, DW), jnp.float32)],
    )(x3, degp, W_fc1, b_fc1.reshape(1, H), W_c1)


def _tc_b(p, degs, b_prev, W_next):
    M = 3 * N_PAD

    def body(p0_ref, p1_ref, d_ref, b_ref, w_ref, xs_ref):
        dis = _dis(d_ref[...])
        h = dis * (p0_ref[...] + p1_ref[...]) + b_ref[...]
        xs_ref[...] = dis * jnp.dot(h, w_ref[...],
                                    preferred_element_type=jnp.float32)

    return pl.pallas_call(
        body,
        grid=(M // BM,),
        in_specs=[_blk(H, 0), _blk(H, M // BM), _blk(DW, 0),
                  _wblk(1, H), _wblk(H, H)],
        out_specs=_blk(H, 0),
        out_shape=jax.ShapeDtypeStruct((M, H), jnp.float32),
    )(p, p, degs, b_prev.reshape(1, H), W_next)


def _tc_c(p, degs, b_c2, W_d1):
    GB = N_PAD // BM

    def body(p01_ref, p11_ref, p02_ref, p12_ref, d0_ref, d1_ref, d2_ref,
             b_ref, w_ref, xs_ref, n2_ref):
        n1 = _dis(d1_ref[...]) * (p01_ref[...] + p11_ref[...]) + b_ref[...]
        n2 = _dis(d2_ref[...]) * (p02_ref[...] + p12_ref[...]) + b_ref[...]
        n2_ref[...] = n2
        xs_ref[...] = _dis(d0_ref[...]) * jnp.dot(
            n1 + n2, w_ref[...], preferred_element_type=jnp.float32)

    return pl.pallas_call(
        body,
        grid=(GB,),
        in_specs=[_blk(H, GB), _blk(H, 4 * GB), _blk(H, 2 * GB),
                  _blk(H, 5 * GB), _blk(DW, 0), _blk(DW, GB), _blk(DW, 2 * GB),
                  _wblk(1, H), _wblk(H, H)],
        out_specs=[_blk(H, 0), _blk(H, 0)],
        out_shape=[jax.ShapeDtypeStruct((N_PAD, H), jnp.float32),
                   jax.ShapeDtypeStruct((N_PAD, H), jnp.float32)],
    )(p, p, p, p, degs, degs, degs, b_c2.reshape(1, H), W_d1)


def _tc_d(p, degs, g, b_d1, W_d2):
    GB = N_PAD // BM

    def body(p0_ref, p1_ref, d_ref, b_ref, w_ref, xs_ref):
        dis = _dis(d_ref[...])
        t = dis * (p0_ref[...] + p1_ref[...]) + b_ref[...]
        xs_ref[...] = dis * jnp.dot(t, w_ref[...],
                                    preferred_element_type=jnp.float32)

    return pl.pallas_call(
        body,
        grid=(GB,),
        in_specs=[_blk(H, 0), _blk(H, GB), _blk(DW, g * GB),
                  _wblk(1, H), _wblk(H, H)],
        out_specs=_blk(H, 0),
        out_shape=jax.ShapeDtypeStruct((N_PAD, H), jnp.float32),
    )(p, p, degs, b_d1.reshape(1, H), W_d2)


def _tc_e(p, degs, gy, gn, b_d2, other, W_d1):
    GB = N_PAD // BM

    def body(p0_ref, p1_ref, dy_ref, dn_ref, b_ref, o_ref, w_ref,
             y_ref, xs_ref):
        y = _dis(dy_ref[...]) * (p0_ref[...] + p1_ref[...]) + b_ref[...]
        y_ref[...] = y
        xs_ref[...] = _dis(dn_ref[...]) * jnp.dot(
            y + o_ref[...], w_ref[...], preferred_element_type=jnp.float32)

    return pl.pallas_call(
        body,
        grid=(GB,),
        in_specs=[_blk(H, 0), _blk(H, GB), _blk(DW, gy * GB),
                  _blk(DW, gn * GB), _wblk(1, H), _blk(H, 0), _wblk(H, H)],
        out_specs=[_blk(H, 0), _blk(H, 0)],
        out_shape=[jax.ShapeDtypeStruct((N_PAD, H), jnp.float32),
                   jax.ShapeDtypeStruct((N_PAD, H), jnp.float32)],
    )(p, p, degs, degs, b_d2.reshape(1, H), other, W_d1)


def _tc_i(p, degs, b_d2, y0, y1, W_fc2, b_fc2):
    GB = N_PAD // BM

    def body(p0_ref, p1_ref, d_ref, b_ref, y0_ref, y1_ref, w_ref, b2_ref,
             y2_ref, fin_ref, loss_ref):
        y2_ref[...] = _dis(d_ref[...]) * (p0_ref[...] + p1_ref[...]) + b_ref[...]
        f = jax.nn.sigmoid(y0_ref[...] * y1_ref[...])
        fin_ref[...] = f
        loss_ref[...] = jnp.dot(f, w_ref[...],
                                preferred_element_type=jnp.float32) + b2_ref[...]

    return pl.pallas_call(
        body,
        grid=(GB,),
        in_specs=[_blk(H, 0), _blk(H, GB), _blk(DW, 2 * GB), _wblk(1, H),
                  _blk(H, 0), _blk(H, 0), _wblk(H, D), _wblk(1, D)],
        out_specs=[_blk(H, 0), _blk(H, 0), _blk(D, 0)],
        out_shape=[jax.ShapeDtypeStruct((N_PAD, H), jnp.float32),
                   jax.ShapeDtypeStruct((N_PAD, H), jnp.float32),
                   jax.ShapeDtypeStruct((N_PAD, D), jnp.float32)],
    )(p, p, degs, b_d2.reshape(1, H), y0, y1, W_fc2, b_fc2.reshape(1, D))


def _edges(ei):
    loop = jnp.arange(N, dtype=jnp.int32)
    fill = jnp.full((E_PAD - E_TOT,), N, jnp.int32)
    src = jnp.concatenate([ei[0].astype(jnp.int32), loop, fill])
    dst = jnp.concatenate([ei[1].astype(jnp.int32), loop, fill])
    return src, dst


def kernel(x0, x1, x2, edge_index0, edge_index1, edge_index2,
           W_fc1, b_fc1, W_c1, b_c1, W_c2, b_c2,
           W_d1, b_d1, W_d2, b_d2, W_fc2, b_fc2):
    edges = [_edges(ei) for ei in (edge_index0, edge_index1, edge_index2)]

    src_c = jnp.concatenate([s + g * N_PAD for g, (s, _) in enumerate(edges)])
    dst_c = jnp.concatenate([d for (_, d) in edges])

    deg_fill = jnp.full((DEG_PAD - 3 * E_TOT,), N, jnp.int32)
    dst_deg = jnp.concatenate(
        [d[:E_TOT] + g * N_PAD for g, (_, d) in enumerate(edges)] + [deg_fill])

    zrows = jnp.zeros((N_PAD // 16, H), jnp.float32)
    zrows_d = jnp.zeros((3 * N_PAD,), jnp.float32)

    degp = _make_deg()(dst_deg, zrows_d).reshape(32, 3 * N_PAD)

    pad = jnp.zeros((N_PAD - N, D), jnp.float32)
    x3 = jnp.concatenate([x0, pad, x1, pad, x2, pad])

    pre3, xs, degs = _tc_a(x3, degp, W_fc1, b_fc1, W_c1)

    spmm3 = _make_spmm(3)
    spmm1 = _make_spmm(1)

    p = spmm3(xs, src_c, dst_c, zrows)
    xs = _tc_b(p, degs, b_c1, W_c2)
    p = spmm3(xs, src_c, dst_c, zrows)
    xs, n2 = _tc_c(p, degs, b_c2, W_d1)

    p = spmm1(xs, edges[0][0], edges[0][1], zrows)
    xs = _tc_d(p, degs, 0, b_d1, W_d2)
    p = spmm1(xs, edges[0][0], edges[0][1], zrows)
    y0, xs = _tc_e(p, degs, 0, 1, b_d2, n2, W_d1)
    p = spmm1(xs, edges[1][0], edges[1][1], zrows)
    xs = _tc_d(p, degs, 1, b_d1, W_d2)
    p = spmm1(xs, edges[1][0], edges[1][1], zrows)
    y1, xs = _tc_e(p, degs, 1, 2, b_d2, y0, W_d1)
    p = spmm1(xs, edges[2][0], edges[2][1], zrows)
    xs = _tc_d(p, degs, 2, b_d1, W_d2)
    p = spmm1(xs, edges[2][0], edges[2][1], zrows)
    y2, fin, loss = _tc_i(p, degs, b_d2, y0, y1, W_fc2, b_fc2)

    pre_x = tuple(pre3.reshape(3, N_PAD, H)[:, :N, :])
    return (pre_x, (y0[:N], y1[:N], y2[:N]), fin[:N], loss[:N])


# speedup vs baseline: 1.5082x; 1.0015x over previous
import functools

import jax
import jax.numpy as jnp
from jax import lax
from jax.experimental import pallas as pl
from jax.experimental.pallas import tpu as pltpu
from jax.experimental.pallas import tpu_sc as plsc

N = 10000
D = 256
H = 128
N_PAD = 10240
BM = 640
E_TOT = 160000 + N
CH = 128
KPG = 48
EPT = KPG * CH
E_PAD = 32 * EPT
DW = 16
KD = 128
DEG_EPT = KD * CH
DEG_PAD = 32 * DEG_EPT


_MESH = plsc.VectorSubcoreMesh(core_axis_name="c", subcore_axis_name="s")


@functools.cache
def _make_spmm(ng):
    @functools.partial(
        pl.kernel,
        out_type=jax.ShapeDtypeStruct((2 * ng * N_PAD, H), jnp.float32),
        mesh=_MESH,
        scratch_types=[
            pltpu.VMEM((EPT,), jnp.int32),
            pltpu.VMEM((CH,), jnp.int32),
            pltpu.VMEM((CH,), jnp.int32),
            pltpu.VMEM((CH, H), jnp.float32),
            pltpu.VMEM((CH, H), jnp.float32),
            pltpu.VMEM_SHARED((N_PAD, H), jnp.float32),
            pltpu.SemaphoreType.DMA,
            pltpu.SemaphoreType.DMA,
            pltpu.SemaphoreType.DMA,
            pltpu.SemaphoreType.DMA,
        ],
    )
    def spmm(table, srcs, dsts, zrows, out, idx_s, idx_da, idx_db,
             rows_a, rows_b, acc, sga, sgb, ssa, ssb):
        c = lax.axis_index("c")
        s = lax.axis_index("s")
        wid = s * 2 + c
        stripe = pl.multiple_of(s * (N_PAD // 16), N_PAD // 16)
        for g in range(ng):
            pltpu.sync_copy(zrows, acc.at[pl.ds(stripe, N_PAD // 16)])
            base0 = pl.multiple_of((g * 32 + wid) * EPT, EPT)
            pltpu.sync_copy(srcs.at[pl.ds(base0, EPT)], idx_s)
            plsc.subcore_barrier()

            @pl.loop(0, KPG // 2)
            def chunk(jj):
                o0 = pl.multiple_of(jj * (2 * CH), 2 * CH)
                o1 = o0 + CH
                pltpu.sync_copy(dsts.at[pl.ds(base0 + o0, CH)], idx_da)
                ga = pltpu.async_copy(
                    table.at[idx_s.at[pl.ds(o0, CH)]], rows_a, sga)
                pltpu.sync_copy(dsts.at[pl.ds(base0 + o1, CH)], idx_db)
                gb = pltpu.async_copy(
                    table.at[idx_s.at[pl.ds(o1, CH)]], rows_b, sgb)
                ga.wait()
                sa = pltpu.async_copy(rows_a, acc.at[idx_da], ssa, add=True)
                gb.wait()
                sb = pltpu.async_copy(rows_b, acc.at[idx_db], ssb, add=True)
                sa.wait()
                sb.wait()

            plsc.subcore_barrier()
            ob = pl.multiple_of((c * ng + g) * N_PAD + stripe, N_PAD // 16)
            pltpu.sync_copy(acc.at[pl.ds(stripe, N_PAD // 16)],
                            out.at[pl.ds(ob, N_PAD // 16)])
            if g + 1 < ng:
                plsc.subcore_barrier()

    return spmm


def _make_deg():
    rows = 3 * N_PAD

    @functools.partial(
        pl.kernel,
        out_type=jax.ShapeDtypeStruct((32 * rows,), jnp.float32),
        mesh=_MESH,
        scratch_types=[
            pltpu.VMEM((CH,), jnp.int32),
            pltpu.VMEM((rows,), jnp.float32),
        ],
        compiler_params=pltpu.CompilerParams(needs_layout_passes=False),
    )
    def deg(dsts, zeros, out, idx_d, acc):
        c = lax.axis_index("c")
        s = lax.axis_index("s")
        wid = s * 2 + c
        pltpu.sync_copy(zeros, acc)
        base0 = wid * DEG_EPT
        one = jnp.full((16,), 1.0, jnp.float32)

        @pl.loop(0, KD)
        def chunk(j):
            b = pl.multiple_of(base0 + j * CH, CH)
            pltpu.sync_copy(dsts.at[pl.ds(b, CH)], idx_d)
            for k in range(CH // 16):
                vec = idx_d[pl.ds(k * 16, 16)]
                plsc.addupdate_scatter(acc, [vec], one)

        ob = pl.multiple_of(wid * rows, rows)
        pltpu.sync_copy(acc, out.at[pl.ds(ob, rows)])

    return deg


def _dis(deg_blk):
    d = deg_blk[:, 0:1]
    return jnp.where(d > 0, 1.0 / jnp.sqrt(d), 0.0)


def _blk(h, off):
    return pl.BlockSpec((BM, h), lambda i, o=off: (i + o, 0))


def _wblk(k, h):
    return pl.BlockSpec((k, h), lambda i: (0, 0))


def _tc_a(x3, degp, W_fc1, b_fc1, W_c1):
    M = 3 * N_PAD

    def body(x_ref, d_ref, w1_ref, b1_ref, wc_ref,
             pre_ref, xs_ref, deg_ref):
        ds = jnp.sum(d_ref[...].T, axis=1, keepdims=True)
        deg_ref[...] = jnp.broadcast_to(ds, (BM, DW))
        pre = jnp.dot(x_ref[...], w1_ref[...],
                      preferred_element_type=jnp.float32) + b1_ref[...]
        pre_ref[...] = pre
        dis = jnp.where(ds > 0, 1.0 / jnp.sqrt(ds), 0.0)
        xs_ref[...] = dis * jnp.dot(pre, wc_ref[...],
                                    preferred_element_type=jnp.float32)

    return pl.pallas_call(
        body,
        grid=(M // BM,),
        in_specs=[_blk(D, 0), pl.BlockSpec((32, BM), lambda i: (0, i)),
                  _wblk(D, H), _wblk(1, H), _wblk(H, H)],
        out_specs=[_blk(H, 0), _blk(H, 0), _blk(DW, 0)],
        out_shape=[jax.ShapeDtypeStruct((M, H), jnp.float32),
                   jax.ShapeDtypeStruct((M, H), jnp.float32),
                   jax.ShapeDtypeStruct((M, DW), jnp.float32)],
    )(x3, degp, W_fc1, b_fc1.reshape(1, H), W_c1)


def _tc_b(p, degs, b_prev, W_next):
    M = 3 * N_PAD

    def body(p0_ref, p1_ref, d_ref, b_ref, w_ref, xs_ref):
        dis = _dis(d_ref[...])
        h = dis * (p0_ref[...] + p1_ref[...]) + b_ref[...]
        xs_ref[...] = dis * jnp.dot(h, w_ref[...],
                                    preferred_element_type=jnp.float32)

    return pl.pallas_call(
        body,
        grid=(M // BM,),
        in_specs=[_blk(H, 0), _blk(H, M // BM), _blk(DW, 0),
                  _wblk(1, H), _wblk(H, H)],
        out_specs=_blk(H, 0),
        out_shape=jax.ShapeDtypeStruct((M, H), jnp.float32),
    )(p, p, degs, b_prev.reshape(1, H), W_next)


def _tc_c(p, degs, b_c2, W_d1):
    GB = N_PAD // BM

    def body(p01_ref, p11_ref, p02_ref, p12_ref, d0_ref, d1_ref, d2_ref,
             b_ref, w_ref, xs_ref, n2_ref):
        n1 = _dis(d1_ref[...]) * (p01_ref[...] + p11_ref[...]) + b_ref[...]
        n2 = _dis(d2_ref[...]) * (p02_ref[...] + p12_ref[...]) + b_ref[...]
        n2_ref[...] = n2
        xs_ref[...] = _dis(d0_ref[...]) * jnp.dot(
            n1 + n2, w_ref[...], preferred_element_type=jnp.float32)

    return pl.pallas_call(
        body,
        grid=(GB,),
        in_specs=[_blk(H, GB), _blk(H, 4 * GB), _blk(H, 2 * GB),
                  _blk(H, 5 * GB), _blk(DW, 0), _blk(DW, GB), _blk(DW, 2 * GB),
                  _wblk(1, H), _wblk(H, H)],
        out_specs=[_blk(H, 0), _blk(H, 0)],
        out_shape=[jax.ShapeDtypeStruct((N_PAD, H), jnp.float32),
                   jax.ShapeDtypeStruct((N_PAD, H), jnp.float32)],
    )(p, p, p, p, degs, degs, degs, b_c2.reshape(1, H), W_d1)


def _tc_d(p, degs, g, b_d1, W_d2):
    GB = N_PAD // BM

    def body(p0_ref, p1_ref, d_ref, b_ref, w_ref, xs_ref):
        dis = _dis(d_ref[...])
        t = dis * (p0_ref[...] + p1_ref[...]) + b_ref[...]
        xs_ref[...] = dis * jnp.dot(t, w_ref[...],
                                    preferred_element_type=jnp.float32)

    return pl.pallas_call(
        body,
        grid=(GB,),
        in_specs=[_blk(H, 0), _blk(H, GB), _blk(DW, g * GB),
                  _wblk(1, H), _wblk(H, H)],
        out_specs=_blk(H, 0),
        out_shape=jax.ShapeDtypeStruct((N_PAD, H), jnp.float32),
    )(p, p, degs, b_d1.reshape(1, H), W_d2)


def _tc_e(p, degs, gy, gn, b_d2, other, W_d1):
    GB = N_PAD // BM

    def body(p0_ref, p1_ref, dy_ref, dn_ref, b_ref, o_ref, w_ref,
             y_ref, xs_ref):
        y = _dis(dy_ref[...]) * (p0_ref[...] + p1_ref[...]) + b_ref[...]
        y_ref[...] = y
        xs_ref[...] = _dis(dn_ref[...]) * jnp.dot(
            y + o_ref[...], w_ref[...], preferred_element_type=jnp.float32)

    return pl.pallas_call(
        body,
        grid=(GB,),
        in_specs=[_blk(H, 0), _blk(H, GB), _blk(DW, gy * GB),
                  _blk(DW, gn * GB), _wblk(1, H), _blk(H, 0), _wblk(H, H)],
        out_specs=[_blk(H, 0), _blk(H, 0)],
        out_shape=[jax.ShapeDtypeStruct((N_PAD, H), jnp.float32),
                   jax.ShapeDtypeStruct((N_PAD, H), jnp.float32)],
    )(p, p, degs, degs, b_d2.reshape(1, H), other, W_d1)


def _tc_i(p, degs, b_d2, y0, y1, W_fc2, b_fc2):
    GB = N_PAD // BM

    def body(p0_ref, p1_ref, d_ref, b_ref, y0_ref, y1_ref, w_ref, b2_ref,
             y2_ref, fin_ref, loss_ref):
        y2_ref[...] = _dis(d_ref[...]) * (p0_ref[...] + p1_ref[...]) + b_ref[...]
        f = jax.nn.sigmoid(y0_ref[...] * y1_ref[...])
        fin_ref[...] = f
        loss_ref[...] = jnp.dot(f, w_ref[...],
                                preferred_element_type=jnp.float32) + b2_ref[...]

    return pl.pallas_call(
        body,
        grid=(GB,),
        in_specs=[_blk(H, 0), _blk(H, GB), _blk(DW, 2 * GB), _wblk(1, H),
                  _blk(H, 0), _blk(H, 0), _wblk(H, D), _wblk(1, D)],
        out_specs=[_blk(H, 0), _blk(H, 0), _blk(D, 0)],
        out_shape=[jax.ShapeDtypeStruct((N_PAD, H), jnp.float32),
                   jax.ShapeDtypeStruct((N_PAD, H), jnp.float32),
                   jax.ShapeDtypeStruct((N_PAD, D), jnp.float32)],
    )(p, p, degs, b_d2.reshape(1, H), y0, y1, W_fc2, b_fc2.reshape(1, D))


def _edges(ei):
    loop = jnp.arange(N, dtype=jnp.int32)
    fill = jnp.full((E_PAD - E_TOT,), N, jnp.int32)
    src = jnp.concatenate([ei[0].astype(jnp.int32), loop, fill])
    dst = jnp.concatenate([ei[1].astype(jnp.int32), loop, fill])
    return src, dst


def kernel(x0, x1, x2, edge_index0, edge_index1, edge_index2,
           W_fc1, b_fc1, W_c1, b_c1, W_c2, b_c2,
           W_d1, b_d1, W_d2, b_d2, W_fc2, b_fc2):
    edges = [_edges(ei) for ei in (edge_index0, edge_index1, edge_index2)]

    src_c = jnp.concatenate([s + g * N_PAD for g, (s, _) in enumerate(edges)])
    dst_c = jnp.concatenate([d for (_, d) in edges])

    deg_fill = jnp.full((DEG_PAD - 3 * E_TOT,), N, jnp.int32)
    dst_deg = jnp.concatenate(
        [d[:E_TOT] + g * N_PAD for g, (_, d) in enumerate(edges)] + [deg_fill])

    zrows = jnp.zeros((N_PAD // 16, H), jnp.float32)
    zrows_d = jnp.zeros((3 * N_PAD,), jnp.float32)

    degp = _make_deg()(dst_deg, zrows_d).reshape(32, 3 * N_PAD)

    pad = jnp.zeros((N_PAD - N, D), jnp.float32)
    x3 = jnp.concatenate([x0, pad, x1, pad, x2, pad])

    pre3, xs, degs = _tc_a(x3, degp, W_fc1, b_fc1, W_c1)

    spmm3 = _make_spmm(3)
    spmm1 = _make_spmm(1)

    p = spmm3(xs, src_c, dst_c, zrows)
    xs = _tc_b(p, degs, b_c1, W_c2)
    p = spmm3(xs, src_c, dst_c, zrows)
    xs, n2 = _tc_c(p, degs, b_c2, W_d1)

    p = spmm1(xs, edges[0][0], edges[0][1], zrows)
    xs = _tc_d(p, degs, 0, b_d1, W_d2)
    p = spmm1(xs, edges[0][0], edges[0][1], zrows)
    y0, xs = _tc_e(p, degs, 0, 1, b_d2, n2, W_d1)
    p = spmm1(xs, edges[1][0], edges[1][1], zrows)
    xs = _tc_d(p, degs, 1, b_d1, W_d2)
    p = spmm1(xs, edges[1][0], edges[1][1], zrows)
    y1, xs = _tc_e(p, degs, 1, 2, b_d2, y0, W_d1)
    p = spmm1(xs, edges[2][0], edges[2][1], zrows)
    xs = _tc_d(p, degs, 2, b_d1, W_d2)
    p = spmm1(xs, edges[2][0], edges[2][1], zrows)
    y2, fin, loss = _tc_i(p, degs, b_d2, y0, y1, W_fc2, b_fc2)

    pre_x = tuple(pre3.reshape(3, N_PAD, H)[:, :N, :])
    return (pre_x, (y0[:N], y1[:N], y2[:N]), fin[:N], loss[:N])
